# baseline (speedup 1.0000x reference)
import functools

import jax
import jax.numpy as jnp
from jax import lax
from jax.experimental import pallas as pl
from jax.experimental.pallas import tpu as pltpu
from jax.experimental.pallas import tpu_sc as plsc

N = 10000
E = 320000
D = 128
H = 8
DH = 16
HOP = 5
ALPHA = 0.15
D_FF = 4 * D

BN = 1000

NC = 2
NS = 16
NW = NC * NS
K = 128
NCH = 80
EWP = NCH * K
EP = NW * EWP
NT = N + 8
KH = 256
NCHH = EWP // KH

DUMPK = 80
_NRCH = N // DUMPK

_mesh = plsc.VectorSubcoreMesh(core_axis_name="c", subcore_axis_name="s")


def _pre_body(x_ref, w_ref, aht_ref, g_ref, b_ref, h_ref, feat_ref, eht_ref):
    x = x_ref[...]
    m = jnp.mean(x, axis=-1, keepdims=True)
    xc = x - m
    var = jnp.mean(xc * xc, axis=-1, keepdims=True)
    h = xc * lax.rsqrt(var + 1e-5) * g_ref[...] + b_ref[...]
    h_ref[...] = h
    feat = lax.dot_general(h, w_ref[...], (((1,), (1,)), ((), ())),
                           preferred_element_type=jnp.float32)
    feat_ref[...] = feat
    eht_ref[...] = jnp.dot(feat, aht_ref[...], preferred_element_type=jnp.float32)


def _pre(x, W_ent, Aht, ln1_g, ln1_b):
    return pl.pallas_call(
        _pre_body,
        grid=(N // BN,),
        in_specs=[
            pl.BlockSpec((BN, D), lambda i: (i, 0)),
            pl.BlockSpec((D, D), lambda i: (0, 0)),
            pl.BlockSpec((D, 2 * H), lambda i: (0, 0)),
            pl.BlockSpec((1, D), lambda i: (0, 0)),
            pl.BlockSpec((1, D), lambda i: (0, 0)),
        ],
        out_specs=[
            pl.BlockSpec((BN, D), lambda i: (i, 0)),
            pl.BlockSpec((BN, D), lambda i: (i, 0)),
            pl.BlockSpec((BN, 2 * H), lambda i: (i, 0)),
        ],
        out_shape=[
            jax.ShapeDtypeStruct((N, D), jnp.float32),
            jax.ShapeDtypeStruct((N, D), jnp.float32),
            jax.ShapeDtypeStruct((N, 2 * H), jnp.float32),
        ],
    )(x, W_ent, Aht, ln1_g.reshape(1, D), ln1_b.reshape(1, D))


def _zero_rows(buf_ref, nrows, width):
    z = jnp.zeros((16,), jnp.float32)

    def body(i, _):
        for t in range(width // 16):
            buf_ref[i, pl.ds(16 * t, 16)] = z
        return 0

    lax.fori_loop(0, nrows, body, 0)


def _spmem_rows_to_hbm(shared_ref, bounce_ref, hbm_view, s):
    def body(t, _):
        k = (s + t * NS) * DUMPK
        pltpu.sync_copy(shared_ref.at[pl.ds(k, DUMPK)], bounce_ref)
        pltpu.sync_copy(bounce_ref, hbm_view.at[pl.ds(k, DUMPK)])
        return 0

    lax.fori_loop(0, (_NRCH - s + NS - 1) // NS, body, 0)


def _zero_shared(shared_ref, bounce_ref, s, width):
    _zero_rows(bounce_ref, DUMPK, width)

    def body(t, _):
        pltpu.sync_copy(bounce_ref, shared_ref.at[pl.ds((s + t * NS) * DUMPK, DUMPK)])
        return 0

    lax.fori_loop(0, (_NRCH - s + NS - 1) // NS, body, 0)

    @pl.when(s == 0)
    def _():
        pltpu.sync_copy(bounce_ref.at[pl.ds(0, NT - N)],
                        shared_ref.at[pl.ds(N, NT - N)])


@functools.partial(
    pl.kernel,
    out_type=[
        jax.ShapeDtypeStruct((EP, 2 * H), jnp.float32),
        jax.ShapeDtypeStruct((NC, NT, 2 * H), jnp.float32),
    ],
    mesh=_mesh,
    compiler_params=pltpu.CompilerParams(use_tc_tiling_on_sc=False),
    scratch_types=[
        pltpu.VMEM((NCH, K), jnp.int32),
        pltpu.VMEM((NCH, K), jnp.int32),
        pltpu.VMEM((K, 2 * H), jnp.float32),
        pltpu.VMEM((K, 2 * H), jnp.float32),
        pltpu.VMEM((K, 2 * H), jnp.float32),
        pltpu.VMEM_SHARED((NT, 2 * H), jnp.float32),
    ],
)
def _edge_logits(src_hbm, dst_hbm, ts_hbm, td_hbm, ee_hbm, dp_hbm,
                 srcv, dstv, rs, rd, eev, dsh):
    c = lax.axis_index("c")
    s = lax.axis_index("s")
    wid = c * NS + s

    _zero_shared(dsh, eev.at[pl.ds(0, DUMPK)], s, 2 * H)
    pltpu.sync_copy(src_hbm.at[wid], srcv)
    pltpu.sync_copy(dst_hbm.at[wid], dstv)
    plsc.subcore_barrier()

    ebase = wid * EWP

    def chunk(j, _):
        pltpu.sync_copy(ts_hbm.at[srcv.at[j]], rs)
        pltpu.sync_copy(td_hbm.at[dstv.at[j]], rd)

        def edge(i, _):
            t = rs[i] + rd[i]
            t = jnp.where(t >= 0.0, t, 0.2 * t)
            eev[i] = jnp.exp(t)
            return 0

        lax.fori_loop(0, K, edge, 0)
        pltpu.sync_copy(eev, dsh.at[dstv.at[j]], add=True)
        pltpu.sync_copy(eev, ee_hbm.at[pl.ds(ebase + j * K, K)])
        return 0

    lax.fori_loop(0, NCH, chunk, 0)
    plsc.subcore_barrier()
    _spmem_rows_to_hbm(dsh, eev.at[pl.ds(0, DUMPK)], dp_hbm.at[c], s)


@functools.partial(
    pl.kernel,
    out_type=jax.ShapeDtypeStruct((EP, 2 * H), jnp.float32),
    mesh=_mesh,
    compiler_params=pltpu.CompilerParams(use_tc_tiling_on_sc=False),
    scratch_types=[
        pltpu.VMEM((NCH, K), jnp.int32),
        pltpu.VMEM((K, 2 * H), jnp.float32),
        pltpu.VMEM((K, 2 * H), jnp.float32),
        pltpu.VMEM((K, 2 * H), jnp.float32),
    ],
)
def _edge_norm(dst_hbm, ee_hbm, p0_hbm, p1_hbm, a_hbm, dstv, r0, r1, eev):
    c = lax.axis_index("c")
    s = lax.axis_index("s")
    wid = c * NS + s
    pltpu.sync_copy(dst_hbm.at[wid], dstv)
    ebase = wid * EWP

    def chunk(j, _):
        pltpu.sync_copy(ee_hbm.at[pl.ds(ebase + j * K, K)], eev)
        pltpu.sync_copy(p0_hbm.at[dstv.at[j]], r0)
        pltpu.sync_copy(p1_hbm.at[dstv.at[j]], r1)

        def edge(i, _):
            eev[i] = eev[i] / (r0[i] + r1[i])
            return 0

        lax.fori_loop(0, K, edge, 0)
        pltpu.sync_copy(eev, a_hbm.at[pl.ds(ebase + j * K, K)])
        return 0

    lax.fori_loop(0, NCH, chunk, 0)


@functools.partial(
    pl.kernel,
    out_type=jax.ShapeDtypeStruct((NC, N, D), jnp.float32),
    mesh=_mesh,
    compiler_params=pltpu.CompilerParams(use_tc_tiling_on_sc=False),
    scratch_types=[
        pltpu.VMEM((NCHH, KH), jnp.int32),
        pltpu.VMEM((KH,), jnp.int32),
        pltpu.VMEM((KH,), jnp.int32),
        pltpu.VMEM((KH, 2 * H), jnp.float32),
        pltpu.VMEM((KH, D), jnp.float32),
        pltpu.VMEM_SHARED((NT, D), jnp.float32),
    ],
)
def _hop(enc_hbm, a_hbm, f_hbm, agg_hbm,
         encv, srcx, dstx, av, rows, ash):
    c = lax.axis_index("c")
    s = lax.axis_index("s")
    wid = c * NS + s

    _zero_shared(ash, rows.at[pl.ds(0, DUMPK)], s, D)
    pltpu.sync_copy(enc_hbm.at[wid], encv)
    plsc.subcore_barrier()

    ebase = wid * EWP

    def chunk(ch, _):
        for t in range(KH // 16):
            sl = pl.ds(16 * t, 16)
            v = encv[ch, sl]
            srcx[sl] = lax.bitwise_and(v, 16383)
            dstx[sl] = lax.shift_right_logical(v, 14)
        pltpu.sync_copy(f_hbm.at[srcx], rows)
        pltpu.sync_copy(a_hbm.at[pl.ds(ebase + ch * KH, KH)], av)

        def edge(i, _):
            arow = av[i]
            for h_ in range(H):
                sl = pl.ds(h_ * DH, DH)
                rows[i, sl] = rows[i, sl] * arow[h_]
            return 0

        lax.fori_loop(0, KH, edge, 0)
        pltpu.sync_copy(rows, ash.at[dstx], add=True)
        return 0

    lax.fori_loop(0, NCHH, chunk, 0)
    plsc.subcore_barrier()
    _spmem_rows_to_hbm(ash, rows.at[pl.ds(0, DUMPK)], agg_hbm.at[c], s)


def _combine_body(p0_ref, p1_ref, f0_ref, o_ref):
    o_ref[...] = ((1.0 - ALPHA) * (p0_ref[...] + p1_ref[...])
                  + ALPHA * f0_ref[...])


def _combine(p0, p1, feat0):
    return pl.pallas_call(
        _combine_body,
        grid=(N // BN,),
        in_specs=[pl.BlockSpec((BN, D), lambda i: (i, 0))] * 3,
        out_specs=pl.BlockSpec((BN, D), lambda i: (i, 0)),
        out_shape=jax.ShapeDtypeStruct((N, D), jnp.float32),
    )(p0, p1, feat0)


def _post_body(p0_ref, p1_ref, f0_ref, h_ref, g_ref, b_ref,
               w1_ref, b1_ref, w2_ref, b2_ref, o_ref):
    f = ((1.0 - ALPHA) * (p0_ref[...] + p1_ref[...]) + ALPHA * f0_ref[...])
    rst = f + h_ref[...]
    m = jnp.mean(rst, axis=-1, keepdims=True)
    xc = rst - m
    var = jnp.mean(xc * xc, axis=-1, keepdims=True)
    h2 = xc * lax.rsqrt(var + 1e-5) * g_ref[...] + b_ref[...]
    z = lax.dot_general(h2, w1_ref[...], (((1,), (1,)), ((), ())),
                        preferred_element_type=jnp.float32) + b1_ref[...]
    z = jnp.maximum(z, 0.0)
    ff = lax.dot_general(z, w2_ref[...], (((1,), (1,)), ((), ())),
                         preferred_element_type=jnp.float32) + b2_ref[...]
    o_ref[...] = ff + rst


def _post(p0, p1, feat0, h, ln2_g, ln2_b, w1, b1, w2, b2):
    return pl.pallas_call(
        _post_body,
        grid=(N // BN,),
        in_specs=[
            pl.BlockSpec((BN, D), lambda i: (i, 0)),
            pl.BlockSpec((BN, D), lambda i: (i, 0)),
            pl.BlockSpec((BN, D), lambda i: (i, 0)),
            pl.BlockSpec((BN, D), lambda i: (i, 0)),
            pl.BlockSpec((1, D), lambda i: (0, 0)),
            pl.BlockSpec((1, D), lambda i: (0, 0)),
            pl.BlockSpec((D_FF, D), lambda i: (0, 0)),
            pl.BlockSpec((1, D_FF), lambda i: (0, 0)),
            pl.BlockSpec((D, D_FF), lambda i: (0, 0)),
            pl.BlockSpec((1, D), lambda i: (0, 0)),
        ],
        out_specs=pl.BlockSpec((BN, D), lambda i: (i, 0)),
        out_shape=jax.ShapeDtypeStruct((N, D), jnp.float32),
    )(p0, p1, feat0, h, ln2_g.reshape(1, D), ln2_b.reshape(1, D),
      w1, b1.reshape(1, D_FF), w2, b2.reshape(1, D))


def kernel(ent_feat, edge_index, ln1_g, ln1_b, W_ent, attn_h, attn_t,
           ln2_g, ln2_b, w1, b1, w2, b2):
    eye = jnp.eye(H, dtype=jnp.float32)
    Ah = (attn_h.reshape(H, DH)[:, :, None] * eye[:, None, :]).reshape(D, H)
    At = (attn_t.reshape(H, DH)[:, :, None] * eye[:, None, :]).reshape(D, H)
    Aht = jnp.concatenate([Ah, At], axis=1)

    h, feat, eht = _pre(ent_feat, W_ent, Aht, ln1_g, ln1_b)
    eh = eht[:, :H]
    et = eht[:, H:]
    t_src = jnp.concatenate([eh, eh], axis=1)
    t_dst = jnp.concatenate([et, et], axis=1)

    npad = EP - E
    srcp = jnp.concatenate([edge_index[0], jnp.zeros((npad,), jnp.int32)])
    dstp = jnp.concatenate([edge_index[1], jnp.full((npad,), N, jnp.int32)])
    src3 = srcp.reshape(NW, NCH, K)
    dst3 = dstp.reshape(NW, NCH, K)
    enc3 = (srcp | (dstp << 14)).reshape(NW, NCHH, KH)

    ee, dp = _edge_logits(src3, dst3, t_src, t_dst)
    a = _edge_norm(dst3, ee, dp[0], dp[1])

    f = feat
    for hop in range(HOP):
        agg = _hop(enc3, a, f)
        if hop < HOP - 1:
            f = _combine(agg[0], agg[1], feat)
    return _post(agg[0], agg[1], feat, h, ln2_g, ln2_b, w1, b1, w2, b2)


# speedup vs baseline: 1.6596x; 1.6596x over previous
import functools

import jax
import jax.numpy as jnp
from jax import lax
from jax.experimental import pallas as pl
from jax.experimental.pallas import tpu as pltpu
from jax.experimental.pallas import tpu_sc as plsc

N = 10000
E = 320000
D = 128
H = 8
DH = 16
HOP = 5
ALPHA = 0.15
D_FF = 4 * D

BN = 1000

NC = 2
NS = 16
NW = NC * NS
EW = E // NW
K = 80
NCH = EW // K

_NRCH = N // K

_mesh = plsc.VectorSubcoreMesh(core_axis_name="c", subcore_axis_name="s")


def _pre_body(x_ref, w_ref, aht_ref, g_ref, b_ref, h_ref, feat_ref, eht_ref):
    x = x_ref[...]
    m = jnp.mean(x, axis=-1, keepdims=True)
    xc = x - m
    var = jnp.mean(xc * xc, axis=-1, keepdims=True)
    h = xc * lax.rsqrt(var + 1e-5) * g_ref[...] + b_ref[...]
    h_ref[...] = h
    feat = lax.dot_general(h, w_ref[...], (((1,), (1,)), ((), ())),
                           preferred_element_type=jnp.float32)
    feat_ref[...] = feat
    eht_ref[...] = jnp.dot(feat, aht_ref[...], preferred_element_type=jnp.float32)


def _pre(x, W_ent, Aht, ln1_g, ln1_b):
    return pl.pallas_call(
        _pre_body,
        grid=(N // BN,),
        in_specs=[
            pl.BlockSpec((BN, D), lambda i: (i, 0)),
            pl.BlockSpec((D, D), lambda i: (0, 0)),
            pl.BlockSpec((D, 2 * H), lambda i: (0, 0)),
            pl.BlockSpec((1, D), lambda i: (0, 0)),
            pl.BlockSpec((1, D), lambda i: (0, 0)),
        ],
        out_specs=[
            pl.BlockSpec((BN, D), lambda i: (i, 0)),
            pl.BlockSpec((BN, D), lambda i: (i, 0)),
            pl.BlockSpec((BN, 2 * H), lambda i: (i, 0)),
        ],
        out_shape=[
            jax.ShapeDtypeStruct((N, D), jnp.float32),
            jax.ShapeDtypeStruct((N, D), jnp.float32),
            jax.ShapeDtypeStruct((N, 2 * H), jnp.float32),
        ],
    )(x, W_ent, Aht, ln1_g.reshape(1, D), ln1_b.reshape(1, D))


def _zero_rows(buf_ref, nrows, width):
    z = jnp.zeros((16,), jnp.float32)

    def body(i, _):
        for t in range(width // 16):
            buf_ref[i, pl.ds(16 * t, 16)] = z
        return 0

    lax.fori_loop(0, nrows, body, 0)


def _spmem_rows_to_hbm(shared_ref, bounce_ref, hbm_view, s):
    def body(t, _):
        k = (s + t * NS) * K
        pltpu.sync_copy(shared_ref.at[pl.ds(k, K)], bounce_ref)
        pltpu.sync_copy(bounce_ref, hbm_view.at[pl.ds(k, K)])
        return 0

    lax.fori_loop(0, (_NRCH - s + NS - 1) // NS, body, 0)


def _zero_shared(shared_ref, bounce_ref, s, width):
    _zero_rows(bounce_ref, K, width)

    def body(t, _):
        pltpu.sync_copy(bounce_ref, shared_ref.at[pl.ds((s + t * NS) * K, K)])
        return 0

    lax.fori_loop(0, (_NRCH - s + NS - 1) // NS, body, 0)


@functools.partial(
    pl.kernel,
    out_type=[
        jax.ShapeDtypeStruct((E, 2 * H), jnp.float32),
        jax.ShapeDtypeStruct((NC, N, 2 * H), jnp.float32),
    ],
    mesh=_mesh,
    compiler_params=pltpu.CompilerParams(use_tc_tiling_on_sc=False),
    scratch_types=[
        pltpu.VMEM((NCH, K), jnp.int32),
        pltpu.VMEM((NCH, K), jnp.int32),
        pltpu.VMEM((K, 2 * H), jnp.float32),
        pltpu.VMEM((K, 2 * H), jnp.float32),
        pltpu.VMEM((K, 2 * H), jnp.float32),
        pltpu.VMEM_SHARED((N, 2 * H), jnp.float32),
    ],
)
def _edge_logits(src_hbm, dst_hbm, ts_hbm, td_hbm, ee_hbm, dp_hbm,
                 srcv, dstv, rs, rd, eev, dsh):
    c = lax.axis_index("c")
    s = lax.axis_index("s")
    wid = c * NS + s

    _zero_shared(dsh, eev, s, 2 * H)
    pltpu.sync_copy(src_hbm.at[wid], srcv)
    pltpu.sync_copy(dst_hbm.at[wid], dstv)
    plsc.subcore_barrier()

    ebase = wid * EW

    def chunk(j, _):
        pltpu.sync_copy(ts_hbm.at[srcv.at[j]], rs)
        pltpu.sync_copy(td_hbm.at[dstv.at[j]], rd)

        def edge(i, _):
            t = rs[i] + rd[i]
            t = jnp.where(t >= 0.0, t, 0.2 * t)
            eev[i] = jnp.exp(t)
            return 0

        lax.fori_loop(0, K, edge, 0)
        pltpu.sync_copy(eev, dsh.at[dstv.at[j]], add=True)
        pltpu.sync_copy(eev, ee_hbm.at[pl.ds(ebase + j * K, K)])
        return 0

    lax.fori_loop(0, NCH, chunk, 0)
    plsc.subcore_barrier()
    _spmem_rows_to_hbm(dsh, eev, dp_hbm.at[c], s)


@functools.partial(
    pl.kernel,
    out_type=jax.ShapeDtypeStruct((E, 2 * H), jnp.float32),
    mesh=_mesh,
    compiler_params=pltpu.CompilerParams(use_tc_tiling_on_sc=False),
    scratch_types=[
        pltpu.VMEM((NCH, K), jnp.int32),
        pltpu.VMEM((K, 2 * H), jnp.float32),
        pltpu.VMEM((K, 2 * H), jnp.float32),
        pltpu.VMEM((K, 2 * H), jnp.float32),
    ],
)
def _edge_norm(dst_hbm, ee_hbm, p0_hbm, p1_hbm, a_hbm, dstv, r0, r1, eev):
    c = lax.axis_index("c")
    s = lax.axis_index("s")
    wid = c * NS + s
    pltpu.sync_copy(dst_hbm.at[wid], dstv)
    ebase = wid * EW

    def chunk(j, _):
        pltpu.sync_copy(ee_hbm.at[pl.ds(ebase + j * K, K)], eev)
        pltpu.sync_copy(p0_hbm.at[dstv.at[j]], r0)
        pltpu.sync_copy(p1_hbm.at[dstv.at[j]], r1)

        def edge(i, _):
            eev[i] = eev[i] / (r0[i] + r1[i])
            return 0

        lax.fori_loop(0, K, edge, 0)
        pltpu.sync_copy(eev, a_hbm.at[pl.ds(ebase + j * K, K)])
        return 0

    lax.fori_loop(0, NCH, chunk, 0)


@functools.partial(
    pl.kernel,
    out_type=jax.ShapeDtypeStruct((NC, N, D), jnp.float32),
    mesh=_mesh,
    compiler_params=pltpu.CompilerParams(use_tc_tiling_on_sc=False),
    scratch_types=[
        pltpu.VMEM((NCH, K), jnp.int32),
        pltpu.VMEM((NCH, K), jnp.int32),
        pltpu.VMEM((K, 2 * H), jnp.float32),
        pltpu.VMEM((K, D), jnp.float32),
        pltpu.VMEM_SHARED((N, D), jnp.float32),
    ],
)
def _hop(src_hbm, dst_hbm, a_hbm, f_hbm, agg_hbm, srcv, dstv, av, rows, ash):
    c = lax.axis_index("c")
    s = lax.axis_index("s")
    wid = c * NS + s

    _zero_shared(ash, rows, s, D)
    pltpu.sync_copy(src_hbm.at[wid], srcv)
    pltpu.sync_copy(dst_hbm.at[wid], dstv)
    plsc.subcore_barrier()

    ebase = wid * EW

    def chunk(j, _):
        pltpu.sync_copy(f_hbm.at[srcv.at[j]], rows)
        pltpu.sync_copy(a_hbm.at[pl.ds(ebase + j * K, K)], av)

        def edge(i, _):
            arow = av[i]
            for h_ in range(H):
                sl = pl.ds(h_ * DH, DH)
                rows[i, sl] = rows[i, sl] * arow[h_]
            return 0

        lax.fori_loop(0, K, edge, 0)
        pltpu.sync_copy(rows, ash.at[dstv.at[j]], add=True)
        return 0

    lax.fori_loop(0, NCH, chunk, 0)
    plsc.subcore_barrier()
    _spmem_rows_to_hbm(ash, rows, agg_hbm.at[c], s)


def _combine_body(p0_ref, p1_ref, f0_ref, o_ref):
    o_ref[...] = ((1.0 - ALPHA) * (p0_ref[...] + p1_ref[...])
                  + ALPHA * f0_ref[...])


def _combine(p0, p1, feat0):
    return pl.pallas_call(
        _combine_body,
        grid=(N // BN,),
        in_specs=[pl.BlockSpec((BN, D), lambda i: (i, 0))] * 3,
        out_specs=pl.BlockSpec((BN, D), lambda i: (i, 0)),
        out_shape=jax.ShapeDtypeStruct((N, D), jnp.float32),
    )(p0, p1, feat0)


def _post_body(p0_ref, p1_ref, f0_ref, h_ref, g_ref, b_ref,
               w1_ref, b1_ref, w2_ref, b2_ref, o_ref):
    f = ((1.0 - ALPHA) * (p0_ref[...] + p1_ref[...]) + ALPHA * f0_ref[...])
    rst = f + h_ref[...]
    m = jnp.mean(rst, axis=-1, keepdims=True)
    xc = rst - m
    var = jnp.mean(xc * xc, axis=-1, keepdims=True)
    h2 = xc * lax.rsqrt(var + 1e-5) * g_ref[...] + b_ref[...]
    z = lax.dot_general(h2, w1_ref[...], (((1,), (1,)), ((), ())),
                        preferred_element_type=jnp.float32) + b1_ref[...]
    z = jnp.maximum(z, 0.0)
    ff = lax.dot_general(z, w2_ref[...], (((1,), (1,)), ((), ())),
                         preferred_element_type=jnp.float32) + b2_ref[...]
    o_ref[...] = ff + rst


def _post(p0, p1, feat0, h, ln2_g, ln2_b, w1, b1, w2, b2):
    return pl.pallas_call(
        _post_body,
        grid=(N // BN,),
        in_specs=[
            pl.BlockSpec((BN, D), lambda i: (i, 0)),
            pl.BlockSpec((BN, D), lambda i: (i, 0)),
            pl.BlockSpec((BN, D), lambda i: (i, 0)),
            pl.BlockSpec((BN, D), lambda i: (i, 0)),
            pl.BlockSpec((1, D), lambda i: (0, 0)),
            pl.BlockSpec((1, D), lambda i: (0, 0)),
            pl.BlockSpec((D_FF, D), lambda i: (0, 0)),
            pl.BlockSpec((1, D_FF), lambda i: (0, 0)),
            pl.BlockSpec((D, D_FF), lambda i: (0, 0)),
            pl.BlockSpec((1, D), lambda i: (0, 0)),
        ],
        out_specs=pl.BlockSpec((BN, D), lambda i: (i, 0)),
        out_shape=jax.ShapeDtypeStruct((N, D), jnp.float32),
    )(p0, p1, feat0, h, ln2_g.reshape(1, D), ln2_b.reshape(1, D),
      w1, b1.reshape(1, D_FF), w2, b2.reshape(1, D))


def kernel(ent_feat, edge_index, ln1_g, ln1_b, W_ent, attn_h, attn_t,
           ln2_g, ln2_b, w1, b1, w2, b2):
    eye = jnp.eye(H, dtype=jnp.float32)
    Ah = (attn_h.reshape(H, DH)[:, :, None] * eye[:, None, :]).reshape(D, H)
    At = (attn_t.reshape(H, DH)[:, :, None] * eye[:, None, :]).reshape(D, H)
    Aht = jnp.concatenate([Ah, At], axis=1)

    h, feat, eht = _pre(ent_feat, W_ent, Aht, ln1_g, ln1_b)
    eh = eht[:, :H]
    et = eht[:, H:]
    t_src = jnp.concatenate([eh, eh], axis=1)
    t_dst = jnp.concatenate([et, et], axis=1)

    src3 = edge_index[0].reshape(NW, NCH, K)
    dst3 = edge_index[1].reshape(NW, NCH, K)

    ee, dp = _edge_logits(src3, dst3, t_src, t_dst)
    a = _edge_norm(dst3, ee, dp[0], dp[1])

    f = feat
    for hop in range(HOP):
        agg = _hop(src3, dst3, a, f)
        if hop < HOP - 1:
            f = _combine(agg[0], agg[1], feat)
    return _post(agg[0], agg[1], feat, h, ln2_g, ln2_b, w1, b1, w2, b2)
